# baseline (speedup 1.0000x reference)
import jax
import jax.numpy as jnp
from jax import lax
from jax.experimental import pallas as pl
from jax.experimental.pallas import tpu as pltpu
from jax.experimental.pallas import tpu_sc as plsc

N = 10000
E = 320000
ET = E + N
D_IN = 128
H1 = 5
C1 = 64
HID = H1 * C1
D_OUT = 128
NEG = 0.2

NC = 2
NS = 16
LANE = 16

EB = 128
EP = 344064
EPH = EP // 2
EW = 8
NPAD = 10240
NBKT = 32
BKT = NPAD // NBKT
CAP = 6144
NBB = CAP // EB
NLIST = NBKT * 2
EPP = NLIST * CAP

EB_P = 1024
NB_P = EPH // EB_P

_f32 = jnp.float32
_i32 = jnp.int32


def _mesh():
  return plsc.VectorSubcoreMesh(
      core_axis_name="c", subcore_axis_name="s", num_cores=NC,
      num_subcores=NS)


def _iota16():
  return lax.iota(_i32, LANE)


_SC_PARAMS = pltpu.CompilerParams(
    needs_layout_passes=False, use_tc_tiling_on_sc=False)


def _t1_body(x_ref, w_ref, asr_ref, adr_ref, xw_ref, as_ref, ad_ref):
  xw = jnp.dot(x_ref[...], w_ref[...], preferred_element_type=_f32)
  xw_ref[...] = xw
  r = xw.shape[0]
  as_ref[...] = (xw * asr_ref[...]).reshape(r, H1, C1).sum(-1)
  ad_ref[...] = (xw * adr_ref[...]).reshape(r, H1, C1).sum(-1)


def _t1(x, W1, a_src1, a_dst1):
  blk = 1000
  return pl.pallas_call(
      _t1_body,
      grid=(N // blk,),
      in_specs=[
          pl.BlockSpec((blk, D_IN), lambda i: (i, 0)),
          pl.BlockSpec((D_IN, HID), lambda i: (0, 0)),
          pl.BlockSpec((1, HID), lambda i: (0, 0)),
          pl.BlockSpec((1, HID), lambda i: (0, 0)),
      ],
      out_specs=[
          pl.BlockSpec((blk, HID), lambda i: (i, 0)),
          pl.BlockSpec((blk, H1), lambda i: (i, 0)),
          pl.BlockSpec((blk, H1), lambda i: (i, 0)),
      ],
      out_shape=[
          jax.ShapeDtypeStruct((N, HID), _f32),
          jax.ShapeDtypeStruct((N, H1), _f32),
          jax.ShapeDtypeStruct((N, H1), _f32),
      ],
  )(x, W1, a_src1.reshape(1, HID), a_dst1.reshape(1, HID))


def _t2_body(d0_ref, d1_ref, b1_ref, w_ref, asr_ref, adr_ref,
             xw_ref, as_ref, ad_ref):
  h0 = jnp.maximum(d0_ref[...] + b1_ref[0, :160], 0.0)
  h1 = jnp.maximum(d1_ref[...] + b1_ref[0, 160:], 0.0)
  xw = (jnp.dot(h0, w_ref[:160, :], preferred_element_type=_f32)
        + jnp.dot(h1, w_ref[160:, :], preferred_element_type=_f32))
  xw_ref[...] = xw
  as_ref[...] = (xw * asr_ref[...]).sum(-1, keepdims=True)
  ad_ref[...] = (xw * adr_ref[...]).sum(-1, keepdims=True)


def _t2(d0, d1, b1, W2, a_src2, a_dst2):
  blk = 1000
  return pl.pallas_call(
      _t2_body,
      grid=(N // blk,),
      in_specs=[
          pl.BlockSpec((blk, 160), lambda i: (i, 0)),
          pl.BlockSpec((blk, 160), lambda i: (i, 0)),
          pl.BlockSpec((1, HID), lambda i: (0, 0)),
          pl.BlockSpec((HID, D_OUT), lambda i: (0, 0)),
          pl.BlockSpec((1, D_OUT), lambda i: (0, 0)),
          pl.BlockSpec((1, D_OUT), lambda i: (0, 0)),
      ],
      out_specs=[
          pl.BlockSpec((blk, D_OUT), lambda i: (i, 0)),
          pl.BlockSpec((blk, 1), lambda i: (i, 0)),
          pl.BlockSpec((blk, 1), lambda i: (i, 0)),
      ],
      out_shape=[
          jax.ShapeDtypeStruct((N, D_OUT), _f32),
          jax.ShapeDtypeStruct((N, 1), _f32),
          jax.ShapeDtypeStruct((N, 1), _f32),
      ],
  )(d0, d1, b1.reshape(1, HID), W2, a_src2.reshape(1, D_OUT),
    a_dst2.reshape(1, D_OUT))


def _inv_body(a_ref, b_ref, o_ref):
  o_ref[...] = 1.0 / (a_ref[...] + b_ref[...] + 1e-16)


def _inv(d2):
  a = d2[0, :N].reshape(625, 128)
  b = d2[1, :N].reshape(625, 128)
  out = pl.pallas_call(
      _inv_body,
      out_shape=jax.ShapeDtypeStruct((625, 128), _f32),
  )(a, b)
  return out.reshape(N, EW)


def _t3_body(q0_ref, q1_ref, b2_ref, o_ref):
  o_ref[...] = (jnp.concatenate([q0_ref[...], q1_ref[...]], axis=1)
                + b2_ref[...])


def _t3(q0, q1, b2):
  blk = 2000
  return pl.pallas_call(
      _t3_body,
      grid=(N // blk,),
      in_specs=[
          pl.BlockSpec((blk, 64), lambda i: (i, 0)),
          pl.BlockSpec((blk, 64), lambda i: (i, 0)),
          pl.BlockSpec((1, D_OUT), lambda i: (0, 0)),
      ],
      out_specs=pl.BlockSpec((blk, D_OUT), lambda i: (i, 0)),
      out_shape=jax.ShapeDtypeStruct((N, D_OUT), _f32),
  )(q0, q1, b2.reshape(1, D_OUT))


def _partition_body(src_hbm, dst_hbm,
                    sp_hbm, dp_hbm, cnt_hbm,
                    sb0, db0, sb1, db1,
                    sa0, da0, sa1, da1, cv, sem0, sem1):
  c = lax.axis_index("c")
  s = lax.axis_index("s")
  iota = _iota16()
  zeros = jnp.zeros((LANE,), _i32)
  lo0 = s * (2 * BKT)
  lo1 = lo0 + BKT
  hb = c * EPH

  def pre(j, _):
    sa0[pl.ds(j * LANE, LANE)] = zeros
    sa1[pl.ds(j * LANE, LANE)] = zeros
    da0[pl.ds(j * LANE, LANE)] = zeros + lo0
    da1[pl.ds(j * LANE, LANE)] = zeros + lo1
    return 0

  lax.fori_loop(0, CAP // LANE, pre, 0)

  bufs = ((sb0, db0, sem0), (sb1, db1, sem1))

  def prep(b, t):
    sb, db, sem = t
    base = pl.multiple_of(hb + b * EB_P, EB_P)
    pltpu.async_copy(src_hbm.at[pl.ds(base, EB_P)], sb, sem)
    pltpu.async_copy(dst_hbm.at[pl.ds(base, EB_P)], db, sem)

  prep(0, bufs[0])

  def batch(b, t, o, carry):
    sb, db, sem = t
    base = pl.multiple_of(hb + b * EB_P, EB_P)

    @pl.when(b + 1 < NB_P)
    def _():
      prep(b + 1, o)

    pltpu.make_async_copy(src_hbm.at[pl.ds(base, EB_P)], sb, sem).wait()
    pltpu.make_async_copy(dst_hbm.at[pl.ds(base, EB_P)], db, sem).wait()
    c0, c1 = carry
    for g in range(EB_P // LANE):
      sv = sb[pl.ds(g * LANE, LANE)]
      dv = db[pl.ds(g * LANE, LANE)]
      k = base + g * LANE + iota
      live = k < ET
      m0 = live & (dv >= lo0) & (dv < lo1)
      m1 = live & (dv >= lo1) & (dv < lo1 + BKT)
      off0 = jnp.minimum(c0, CAP - LANE)
      plsc.store_compressed(sa0.at[pl.ds(off0, LANE)], sv, mask=m0)
      plsc.store_compressed(da0.at[pl.ds(off0, LANE)], dv, mask=m0)
      c0 = c0 + plsc.all_reduce_population_count(m0)[0]
      off1 = jnp.minimum(c1, CAP - LANE)
      plsc.store_compressed(sa1.at[pl.ds(off1, LANE)], sv, mask=m1)
      plsc.store_compressed(da1.at[pl.ds(off1, LANE)], dv, mask=m1)
      c1 = c1 + plsc.all_reduce_population_count(m1)[0]
    return c0, c1

  def pair(i, carry):
    carry = batch(2 * i, bufs[0], bufs[1], carry)
    carry = batch(2 * i + 1, bufs[1], bufs[0], carry)
    return carry

  c0, c1 = lax.fori_loop(0, NB_P // 2, pair, (0, 0))

  for bi, (sa, da, cnt) in enumerate(((sa0, da0, c0), (sa1, da1, c1))):
    lid = (2 * s + bi) * 2 + c
    off = pl.multiple_of(lid * CAP, EB)
    pltpu.sync_copy(sa, sp_hbm.at[pl.ds(off, CAP)])
    pltpu.sync_copy(da, dp_hbm.at[pl.ds(off, CAP)])
    cv[0, :] = jnp.zeros((LANE,), _i32) + cnt
    pltpu.sync_copy(cv, cnt_hbm.at[pl.ds(lid, 1)])


_partition = pl.kernel(
    _partition_body,
    out_type=[
        jax.ShapeDtypeStruct((EPP,), _i32),
        jax.ShapeDtypeStruct((EPP,), _i32),
        jax.ShapeDtypeStruct((NLIST, LANE), _i32),
    ],
    mesh=_mesh(),
    compiler_params=_SC_PARAMS,
    scratch_types=[
        pltpu.VMEM((EB_P,), _i32),
        pltpu.VMEM((EB_P,), _i32),
        pltpu.VMEM((EB_P,), _i32),
        pltpu.VMEM((EB_P,), _i32),
        pltpu.VMEM((CAP,), _i32),
        pltpu.VMEM((CAP,), _i32),
        pltpu.VMEM((CAP,), _i32),
        pltpu.VMEM((CAP,), _i32),
        pltpu.VMEM((1, LANE), _i32),
        pltpu.SemaphoreType.DMA,
        pltpu.SemaphoreType.DMA,
    ],
)


def _make_pass_a(heads):
  tab = N * heads

  def body(sp_hbm, dp_hbm, cnt_hbm, as_hbm, ad_hbm, c_hbm,
           ee_hbm, den_hbm,
           as_v, ad_v, c_v, cntv, srcb, dstb, stage, den_sh):
    c = lax.axis_index("c")
    s = lax.axis_index("s")
    pltpu.sync_copy(as_hbm, as_v)
    pltpu.sync_copy(ad_hbm, ad_v)
    pltpu.sync_copy(c_hbm, c_v)

    zeros = jnp.zeros((LANE,), _f32)
    iota = _iota16()
    zrow_r = iota // EW
    zrow_c = iota % EW

    def zrow(j, _):
      plsc.store_scatter(stage, [2 * j + zrow_r, zrow_c], zeros)
      return 0

    lax.fori_loop(0, EB // 2, zrow, 0)

    r0 = s * (NPAD // NS)
    for j in range(5):
      pltpu.sync_copy(stage, den_sh.at[pl.ds(r0 + j * EB, EB)])
    plsc.subcore_barrier()

    for bi in range(2):
      lid = (2 * s + bi) * 2 + c
      loff = pl.multiple_of(lid * CAP, EB)
      pltpu.sync_copy(cnt_hbm.at[pl.ds(lid, 1)], cntv)
      cnt = cntv[0, :]

      def batch(b, _):
        base = pl.multiple_of(loff + b * EB, EB)
        pltpu.sync_copy(sp_hbm.at[pl.ds(base, EB)], srcb)
        pltpu.sync_copy(dp_hbm.at[pl.ds(base, EB)], dstb)
        for g in range(EB // LANE):
          sv = srcb[pl.ds(g * LANE, LANE)]
          dv = dstb[pl.ds(g * LANE, LANE)]
          kl = b * EB + g * LANE + iota
          live = kl < cnt
          for h in range(heads):
            av = plsc.load_gather(as_v, [sv * heads + h])
            bv = plsc.load_gather(ad_v, [dv * heads + h])
            e = av + bv
            e = jnp.maximum(e, NEG * e)
            ee = jnp.exp(e - c_v[h, :])
            ee = jnp.where(live, ee, 0.0)
            plsc.store_scatter(
                stage, [g * LANE + iota, jnp.full((LANE,), h, _i32)], ee)
        pltpu.sync_copy(stage, ee_hbm.at[pl.ds(base, EB)])
        pltpu.sync_copy(stage, den_sh.at[dstb], add=True)
        return 0

      lax.fori_loop(0, NBB, batch, 0)

    plsc.subcore_barrier()

    @pl.when(c == 0)
    def _():
      pltpu.sync_copy(den_sh.at[pl.ds(r0, NPAD // NS)],
                      den_hbm.at[0, pl.ds(r0, NPAD // NS)])

    @pl.when(c == 1)
    def _():
      pltpu.sync_copy(den_sh.at[pl.ds(r0, NPAD // NS)],
                      den_hbm.at[1, pl.ds(r0, NPAD // NS)])

  return pl.kernel(
      body,
      out_type=[
          jax.ShapeDtypeStruct((EPP, EW), _f32),
          jax.ShapeDtypeStruct((2, NPAD, EW), _f32),
      ],
      mesh=_mesh(),
      compiler_params=_SC_PARAMS,
      scratch_types=[
          pltpu.VMEM((tab,), _f32),
          pltpu.VMEM((tab,), _f32),
          pltpu.VMEM((heads, LANE), _f32),
          pltpu.VMEM((1, LANE), _i32),
          pltpu.VMEM((EB,), _i32),
          pltpu.VMEM((EB,), _i32),
          pltpu.VMEM((EB, EW), _f32),
          pltpu.VMEM_SHARED((NPAD, EW), _f32),
      ],
  )


def _make_pass_b(heads, ch, hmap0, hmap1, D=3, LOOK=2):
  nv = ch // LANE
  ng = EB // LANE
  NB2 = 2 * NBB

  def body(sp_hbm, dp_hbm, ee_hbm, inv_hbm, xw_hbm,
           out0_hbm, out1_hbm, *rest):
    bufs = tuple(tuple(rest[9 * k: 9 * (k + 1)]) for k in range(D))
    wb, acc = rest[9 * D:]
    c = lax.axis_index("c")
    s = lax.axis_index("s")

    zeros = jnp.zeros((LANE,), _f32)
    iota = _iota16()
    coff = c * N

    def prep_and_gather(base2, b, t):
      (srcb, dstb, gidx, eeb, invb, rows, sem_i, sem_d, sem_g) = t
      base = pl.multiple_of(base2 + b * EB, EB)
      pltpu.async_copy(sp_hbm.at[pl.ds(base, EB)], srcb, sem_i)
      pltpu.async_copy(dp_hbm.at[pl.ds(base, EB)], dstb, sem_i)
      pltpu.async_copy(ee_hbm.at[pl.ds(base, EB)], eeb, sem_d)
      pltpu.make_async_copy(
          sp_hbm.at[pl.ds(base, EB)], srcb, sem_i).wait()
      pltpu.make_async_copy(
          dp_hbm.at[pl.ds(base, EB)], dstb, sem_i).wait()
      for g in range(ng):
        sv = srcb[pl.ds(g * LANE, LANE)]
        gidx[pl.ds(g * LANE, LANE)] = sv + coff
      pltpu.async_copy(xw_hbm.at[gidx], rows, sem_g)
      pltpu.async_copy(inv_hbm.at[dstb], invb, sem_d)

    def zacc(j, _):
      for v in range(nv):
        acc[j, pl.ds(v * LANE, LANE)] = zeros
      return 0

    def bucket(bi, _):
      bkt = 2 * s + bi
      lo = bkt * BKT
      base2 = pl.multiple_of(bkt * (2 * CAP), EB)

      lax.fori_loop(0, BKT, zacc, 0)

      for k in range(LOOK):
        prep_and_gather(base2, k, bufs[k])

      def phase(b, t, t_next):
        (srcb, dstb, gidx, eeb, invb, rows, sem_i, sem_d, sem_g) = t
        base = pl.multiple_of(base2 + b * EB, EB)

        @pl.when(b + LOOK < NB2)
        def _():
          prep_and_gather(base2, b + LOOK, t_next)

        pltpu.make_async_copy(xw_hbm.at[gidx], rows, sem_g).wait()
        pltpu.make_async_copy(
            ee_hbm.at[pl.ds(base, EB)], eeb, sem_d).wait()
        pltpu.make_async_copy(inv_hbm.at[dstb], invb, sem_d).wait()

        for g in range(ng):
          for h in range(heads):
            eev = plsc.load_gather(
                eeb, [g * LANE + iota, jnp.full((LANE,), h, _i32)])
            iv = plsc.load_gather(
                invb, [g * LANE + iota, jnp.full((LANE,), h, _i32)])
            wb[h, pl.ds(g * LANE, LANE)] = eev * iv

        def accgroup(hmap):
          heads_used = sorted(set(hmap))

          def run(g, _):
            goff = pl.multiple_of(g * LANE, LANE)
            dl = dstb[pl.ds(goff, LANE)] - lo
            wvs = {h: wb[h, pl.ds(goff, LANE)] for h in heads_used}
            for jj in range(LANE):
              j = goff + jj
              d = dl[jj]
              ws = {h: wvs[h][jj] for h in heads_used}
              for v in range(nv):
                acc[d, pl.ds(v * LANE, LANE)] = (
                    acc[d, pl.ds(v * LANE, LANE)]
                    + rows[j, pl.ds(v * LANE, LANE)] * ws[hmap[v]])
            return 0

          lax.fori_loop(0, ng, run, 0)

        @pl.when(c == 0)
        def _():
          accgroup(hmap0)

        @pl.when(c == 1)
        def _():
          accgroup(hmap1)

      def ring(i, _):
        for q in range(D):
          b = D * i + q
          phase(b, bufs[q], bufs[(q + LOOK) % D])
        return 0

      lax.fori_loop(0, NB2 // D, ring, 0)

      @pl.when(c == 0)
      def _():
        pltpu.sync_copy(acc, out0_hbm.at[pl.ds(lo, BKT)])

      @pl.when(c == 1)
      def _():
        pltpu.sync_copy(acc, out1_hbm.at[pl.ds(lo, BKT)])

      return 0

    lax.fori_loop(0, 2, bucket, 0)

  dbuf = [
      pltpu.VMEM((EB,), _i32),
      pltpu.VMEM((EB,), _i32),
      pltpu.VMEM((EB,), _i32),
      pltpu.VMEM((EB, EW), _f32),
      pltpu.VMEM((EB, EW), _f32),
      pltpu.VMEM((EB, ch), _f32),
      pltpu.SemaphoreType.DMA,
      pltpu.SemaphoreType.DMA,
      pltpu.SemaphoreType.DMA,
  ]
  return pl.kernel(
      body,
      out_type=[jax.ShapeDtypeStruct((NPAD, ch), _f32)] * 2,
      mesh=_mesh(),
      compiler_params=_SC_PARAMS,
      scratch_types=(
          dbuf * D
          + [pltpu.VMEM((8, EB), _f32),
             pltpu.VMEM((BKT, ch), _f32)]),
  )


_HMAP_L1 = tuple(LANE * v // C1 for v in range(10))
_HMAP_L1_HI = tuple((160 + LANE * v) // C1 for v in range(10))
_HMAP_L2 = (0, 0, 0, 0)

_pass_a1 = _make_pass_a(H1)
_pass_a2 = _make_pass_a(1)
_pass_b1 = _make_pass_b(H1, 160, _HMAP_L1, _HMAP_L1_HI)
_pass_b2 = _make_pass_b(1, 64, _HMAP_L2, _HMAP_L2)


def _layer(sp, dp, cnts, xw, alpha_s, alpha_d, pass_a, pass_b, heads, ch):
  m = jnp.max(alpha_s, axis=0) + jnp.max(alpha_d, axis=0)
  cvec = jnp.maximum(m, NEG * m)
  cb = jnp.broadcast_to(cvec[:, None], (heads, LANE)).astype(_f32)
  ee, den = pass_a(sp, dp, cnts, alpha_s.ravel(), alpha_d.ravel(), cb)
  inv = _inv(den)
  xw_all = jnp.concatenate([xw[:, :ch], xw[:, ch:]], axis=0)
  o0, o1 = pass_b(sp, dp, ee, inv, xw_all)
  return o0[:N], o1[:N]


def kernel(x, edge_index, W1, a_src1, a_dst1, b1, W2, a_src2, a_dst2, b2):
  ei = edge_index.astype(_i32)
  loop = jnp.arange(N, dtype=_i32)
  padz = jnp.zeros((EP - ET,), _i32)
  src = jnp.concatenate([ei[0], loop, padz])
  dst = jnp.concatenate([ei[1], loop, padz])
  sp, dp, cnts = _partition(src, dst)

  xw1, as1, ad1 = _t1(x, W1, a_src1, a_dst1)
  d0, d1 = _layer(sp, dp, cnts, xw1, as1, ad1, _pass_a1, _pass_b1, H1, 160)
  xw2, as2, ad2 = _t2(d0, d1, b1, W2, a_src2, a_dst2)
  q0, q1 = _layer(sp, dp, cnts, xw2, as2, ad2, _pass_a2, _pass_b2, 1, 64)
  return _t3(q0, q1, b2)


# speedup vs baseline: 1.0010x; 1.0010x over previous
import jax
import jax.numpy as jnp
from jax import lax
from jax.experimental import pallas as pl
from jax.experimental.pallas import tpu as pltpu
from jax.experimental.pallas import tpu_sc as plsc

N = 10000
E = 320000
ET = E + N
D_IN = 128
H1 = 5
C1 = 64
HID = H1 * C1
D_OUT = 128
NEG = 0.2

NC = 2
NS = 16
LANE = 16

EB = 128
EP = 344064
EPH = EP // 2
EW = 8
NPAD = 10240
NBKT = 32
BKT = NPAD // NBKT
CAP = 6144
NBB = CAP // EB
NLIST = NBKT * 2
EPP = NLIST * CAP

EB_P = 1024
NB_P = EPH // EB_P

_f32 = jnp.float32
_i32 = jnp.int32


def _mesh():
  return plsc.VectorSubcoreMesh(
      core_axis_name="c", subcore_axis_name="s", num_cores=NC,
      num_subcores=NS)


def _iota16():
  return lax.iota(_i32, LANE)


_SC_PARAMS = pltpu.CompilerParams(
    needs_layout_passes=False, use_tc_tiling_on_sc=False)


def _t1_body(x_ref, w_ref, asr_ref, adr_ref, xw_ref, as_ref, ad_ref):
  xw = jnp.dot(x_ref[...], w_ref[...], preferred_element_type=_f32)
  xw_ref[...] = xw
  r = xw.shape[0]
  as_ref[...] = (xw * asr_ref[...]).reshape(r, H1, C1).sum(-1)
  ad_ref[...] = (xw * adr_ref[...]).reshape(r, H1, C1).sum(-1)


def _t1(x, W1, a_src1, a_dst1):
  blk = 1000
  return pl.pallas_call(
      _t1_body,
      grid=(N // blk,),
      in_specs=[
          pl.BlockSpec((blk, D_IN), lambda i: (i, 0)),
          pl.BlockSpec((D_IN, HID), lambda i: (0, 0)),
          pl.BlockSpec((1, HID), lambda i: (0, 0)),
          pl.BlockSpec((1, HID), lambda i: (0, 0)),
      ],
      out_specs=[
          pl.BlockSpec((blk, HID), lambda i: (i, 0)),
          pl.BlockSpec((blk, H1), lambda i: (i, 0)),
          pl.BlockSpec((blk, H1), lambda i: (i, 0)),
      ],
      out_shape=[
          jax.ShapeDtypeStruct((N, HID), _f32),
          jax.ShapeDtypeStruct((N, H1), _f32),
          jax.ShapeDtypeStruct((N, H1), _f32),
      ],
  )(x, W1, a_src1.reshape(1, HID), a_dst1.reshape(1, HID))


def _t2_body(d0_ref, d1_ref, b1_ref, w_ref, asr_ref, adr_ref,
             xw_ref, as_ref, ad_ref):
  h0 = jnp.maximum(d0_ref[...] + b1_ref[0, :160], 0.0)
  h1 = jnp.maximum(d1_ref[...] + b1_ref[0, 160:], 0.0)
  xw = (jnp.dot(h0, w_ref[:160, :], preferred_element_type=_f32)
        + jnp.dot(h1, w_ref[160:, :], preferred_element_type=_f32))
  xw_ref[...] = xw
  as_ref[...] = (xw * asr_ref[...]).sum(-1, keepdims=True)
  ad_ref[...] = (xw * adr_ref[...]).sum(-1, keepdims=True)


def _t2(d0, d1, b1, W2, a_src2, a_dst2):
  blk = 1000
  return pl.pallas_call(
      _t2_body,
      grid=(N // blk,),
      in_specs=[
          pl.BlockSpec((blk, 160), lambda i: (i, 0)),
          pl.BlockSpec((blk, 160), lambda i: (i, 0)),
          pl.BlockSpec((1, HID), lambda i: (0, 0)),
          pl.BlockSpec((HID, D_OUT), lambda i: (0, 0)),
          pl.BlockSpec((1, D_OUT), lambda i: (0, 0)),
          pl.BlockSpec((1, D_OUT), lambda i: (0, 0)),
      ],
      out_specs=[
          pl.BlockSpec((blk, D_OUT), lambda i: (i, 0)),
          pl.BlockSpec((blk, 1), lambda i: (i, 0)),
          pl.BlockSpec((blk, 1), lambda i: (i, 0)),
      ],
      out_shape=[
          jax.ShapeDtypeStruct((N, D_OUT), _f32),
          jax.ShapeDtypeStruct((N, 1), _f32),
          jax.ShapeDtypeStruct((N, 1), _f32),
      ],
  )(d0, d1, b1.reshape(1, HID), W2, a_src2.reshape(1, D_OUT),
    a_dst2.reshape(1, D_OUT))


def _inv_body(a_ref, b_ref, o_ref):
  o_ref[...] = 1.0 / (a_ref[...] + b_ref[...] + 1e-16)


def _inv(d2):
  a = d2[0, :N].reshape(625, 128)
  b = d2[1, :N].reshape(625, 128)
  out = pl.pallas_call(
      _inv_body,
      out_shape=jax.ShapeDtypeStruct((625, 128), _f32),
  )(a, b)
  return out.reshape(N, EW)


def _t3_body(q0_ref, q1_ref, b2_ref, o_ref):
  o_ref[...] = (jnp.concatenate([q0_ref[...], q1_ref[...]], axis=1)
                + b2_ref[...])


def _t3(q0, q1, b2):
  blk = 2000
  return pl.pallas_call(
      _t3_body,
      grid=(N // blk,),
      in_specs=[
          pl.BlockSpec((blk, 64), lambda i: (i, 0)),
          pl.BlockSpec((blk, 64), lambda i: (i, 0)),
          pl.BlockSpec((1, D_OUT), lambda i: (0, 0)),
      ],
      out_specs=pl.BlockSpec((blk, D_OUT), lambda i: (i, 0)),
      out_shape=jax.ShapeDtypeStruct((N, D_OUT), _f32),
  )(q0, q1, b2.reshape(1, D_OUT))


def _partition_body(src_hbm, dst_hbm,
                    sp_hbm, dp_hbm, cnt_hbm,
                    sb0, db0, sb1, db1,
                    sa0, da0, sa1, da1, cv, sem0, sem1):
  c = lax.axis_index("c")
  s = lax.axis_index("s")
  iota = _iota16()
  zeros = jnp.zeros((LANE,), _i32)
  lo0 = s * (2 * BKT)
  lo1 = lo0 + BKT
  hb = c * EPH

  def pre(j, _):
    sa0[pl.ds(j * LANE, LANE)] = zeros
    sa1[pl.ds(j * LANE, LANE)] = zeros
    da0[pl.ds(j * LANE, LANE)] = zeros + lo0
    da1[pl.ds(j * LANE, LANE)] = zeros + lo1
    return 0

  lax.fori_loop(0, CAP // LANE, pre, 0)

  bufs = ((sb0, db0, sem0), (sb1, db1, sem1))

  def prep(b, t):
    sb, db, sem = t
    base = pl.multiple_of(hb + b * EB_P, EB_P)
    pltpu.async_copy(src_hbm.at[pl.ds(base, EB_P)], sb, sem)
    pltpu.async_copy(dst_hbm.at[pl.ds(base, EB_P)], db, sem)

  prep(0, bufs[0])

  def batch(b, t, o, carry):
    sb, db, sem = t
    base = pl.multiple_of(hb + b * EB_P, EB_P)

    @pl.when(b + 1 < NB_P)
    def _():
      prep(b + 1, o)

    pltpu.make_async_copy(src_hbm.at[pl.ds(base, EB_P)], sb, sem).wait()
    pltpu.make_async_copy(dst_hbm.at[pl.ds(base, EB_P)], db, sem).wait()
    c0, c1 = carry
    for g in range(EB_P // LANE):
      sv = sb[pl.ds(g * LANE, LANE)]
      dv = db[pl.ds(g * LANE, LANE)]
      k = base + g * LANE + iota
      live = k < ET
      m0 = live & (dv >= lo0) & (dv < lo1)
      m1 = live & (dv >= lo1) & (dv < lo1 + BKT)
      off0 = jnp.minimum(c0, CAP - LANE)
      plsc.store_compressed(sa0.at[pl.ds(off0, LANE)], sv, mask=m0)
      plsc.store_compressed(da0.at[pl.ds(off0, LANE)], dv, mask=m0)
      c0 = c0 + plsc.all_reduce_population_count(m0)[0]
      off1 = jnp.minimum(c1, CAP - LANE)
      plsc.store_compressed(sa1.at[pl.ds(off1, LANE)], sv, mask=m1)
      plsc.store_compressed(da1.at[pl.ds(off1, LANE)], dv, mask=m1)
      c1 = c1 + plsc.all_reduce_population_count(m1)[0]
    return c0, c1

  def pair(i, carry):
    carry = batch(2 * i, bufs[0], bufs[1], carry)
    carry = batch(2 * i + 1, bufs[1], bufs[0], carry)
    return carry

  c0, c1 = lax.fori_loop(0, NB_P // 2, pair, (0, 0))

  for bi, (sa, da, cnt) in enumerate(((sa0, da0, c0), (sa1, da1, c1))):
    lid = (2 * s + bi) * 2 + c
    off = pl.multiple_of(lid * CAP, EB)
    pltpu.sync_copy(sa, sp_hbm.at[pl.ds(off, CAP)])
    pltpu.sync_copy(da, dp_hbm.at[pl.ds(off, CAP)])
    cv[0, :] = jnp.zeros((LANE,), _i32) + cnt
    pltpu.sync_copy(cv, cnt_hbm.at[pl.ds(lid, 1)])


_partition = pl.kernel(
    _partition_body,
    out_type=[
        jax.ShapeDtypeStruct((EPP,), _i32),
        jax.ShapeDtypeStruct((EPP,), _i32),
        jax.ShapeDtypeStruct((NLIST, LANE), _i32),
    ],
    mesh=_mesh(),
    compiler_params=_SC_PARAMS,
    scratch_types=[
        pltpu.VMEM((EB_P,), _i32),
        pltpu.VMEM((EB_P,), _i32),
        pltpu.VMEM((EB_P,), _i32),
        pltpu.VMEM((EB_P,), _i32),
        pltpu.VMEM((CAP,), _i32),
        pltpu.VMEM((CAP,), _i32),
        pltpu.VMEM((CAP,), _i32),
        pltpu.VMEM((CAP,), _i32),
        pltpu.VMEM((1, LANE), _i32),
        pltpu.SemaphoreType.DMA,
        pltpu.SemaphoreType.DMA,
    ],
)


def _make_pass_a(heads):
  tab = N * heads

  def body(sp_hbm, dp_hbm, cnt_hbm, as_hbm, ad_hbm, c_hbm,
           ee_hbm, den_hbm,
           as_v, ad_v, c_v, cntv, srcb, dstb, stage, den_sh):
    c = lax.axis_index("c")
    s = lax.axis_index("s")
    pltpu.sync_copy(as_hbm, as_v)
    pltpu.sync_copy(ad_hbm, ad_v)
    pltpu.sync_copy(c_hbm, c_v)

    zeros = jnp.zeros((LANE,), _f32)
    iota = _iota16()
    zrow_r = iota // EW
    zrow_c = iota % EW

    def zrow(j, _):
      plsc.store_scatter(stage, [2 * j + zrow_r, zrow_c], zeros)
      return 0

    lax.fori_loop(0, EB // 2, zrow, 0)

    r0 = s * (NPAD // NS)
    for j in range(5):
      pltpu.sync_copy(stage, den_sh.at[pl.ds(r0 + j * EB, EB)])
    plsc.subcore_barrier()

    for bi in range(2):
      lid = (2 * s + bi) * 2 + c
      loff = pl.multiple_of(lid * CAP, EB)
      pltpu.sync_copy(cnt_hbm.at[pl.ds(lid, 1)], cntv)
      cnt = cntv[0, :]

      def batch(b, _):
        base = pl.multiple_of(loff + b * EB, EB)
        pltpu.sync_copy(sp_hbm.at[pl.ds(base, EB)], srcb)
        pltpu.sync_copy(dp_hbm.at[pl.ds(base, EB)], dstb)
        for g in range(EB // LANE):
          sv = srcb[pl.ds(g * LANE, LANE)]
          dv = dstb[pl.ds(g * LANE, LANE)]
          kl = b * EB + g * LANE + iota
          live = kl < cnt
          for h in range(heads):
            av = plsc.load_gather(as_v, [sv * heads + h])
            bv = plsc.load_gather(ad_v, [dv * heads + h])
            e = av + bv
            e = jnp.maximum(e, NEG * e)
            ee = jnp.exp(e - c_v[h, :])
            ee = jnp.where(live, ee, 0.0)
            plsc.store_scatter(
                stage, [g * LANE + iota, jnp.full((LANE,), h, _i32)], ee)
        pltpu.sync_copy(stage, ee_hbm.at[pl.ds(base, EB)])
        pltpu.sync_copy(stage, den_sh.at[dstb], add=True)
        return 0

      lax.fori_loop(0, NBB, batch, 0)

    plsc.subcore_barrier()

    @pl.when(c == 0)
    def _():
      pltpu.sync_copy(den_sh.at[pl.ds(r0, NPAD // NS)],
                      den_hbm.at[0, pl.ds(r0, NPAD // NS)])

    @pl.when(c == 1)
    def _():
      pltpu.sync_copy(den_sh.at[pl.ds(r0, NPAD // NS)],
                      den_hbm.at[1, pl.ds(r0, NPAD // NS)])

  return pl.kernel(
      body,
      out_type=[
          jax.ShapeDtypeStruct((EPP, EW), _f32),
          jax.ShapeDtypeStruct((2, NPAD, EW), _f32),
      ],
      mesh=_mesh(),
      compiler_params=_SC_PARAMS,
      scratch_types=[
          pltpu.VMEM((tab,), _f32),
          pltpu.VMEM((tab,), _f32),
          pltpu.VMEM((heads, LANE), _f32),
          pltpu.VMEM((1, LANE), _i32),
          pltpu.VMEM((EB,), _i32),
          pltpu.VMEM((EB,), _i32),
          pltpu.VMEM((EB, EW), _f32),
          pltpu.VMEM_SHARED((NPAD, EW), _f32),
      ],
  )


def _make_pass_b(heads, ch, hmap0, hmap1, D=3, LOOK=2):
  nv = ch // LANE
  ng = EB // LANE
  NB2 = 2 * NBB

  def body(sp_hbm, dp_hbm, ee_hbm, inv_hbm, xw_hbm,
           out0_hbm, out1_hbm, *rest):
    bufs = tuple(tuple(rest[9 * k: 9 * (k + 1)]) for k in range(D))
    wb, acc = rest[9 * D:]
    c = lax.axis_index("c")
    s = lax.axis_index("s")

    zeros = jnp.zeros((LANE,), _f32)
    iota = _iota16()
    coff = c * N

    def prep_and_gather(base2, b, t):
      (srcb, dstb, gidx, eeb, invb, rows, sem_i, sem_d, sem_g) = t
      base = pl.multiple_of(base2 + b * EB, EB)
      pltpu.async_copy(sp_hbm.at[pl.ds(base, EB)], srcb, sem_i)
      pltpu.async_copy(dp_hbm.at[pl.ds(base, EB)], dstb, sem_i)
      pltpu.async_copy(ee_hbm.at[pl.ds(base, EB)], eeb, sem_d)
      pltpu.make_async_copy(
          sp_hbm.at[pl.ds(base, EB)], srcb, sem_i).wait()
      pltpu.make_async_copy(
          dp_hbm.at[pl.ds(base, EB)], dstb, sem_i).wait()
      for g in range(ng):
        sv = srcb[pl.ds(g * LANE, LANE)]
        gidx[pl.ds(g * LANE, LANE)] = sv + coff
      pltpu.async_copy(xw_hbm.at[gidx], rows, sem_g)
      pltpu.async_copy(inv_hbm.at[dstb], invb, sem_d)

    def zacc(j, _):
      acc[pl.ds(j * LANE, LANE)] = zeros
      return 0

    def bucket(bi, _):
      bkt = 2 * s + bi
      lo = bkt * BKT
      base2 = pl.multiple_of(bkt * (2 * CAP), EB)

      lax.fori_loop(0, BKT * nv, zacc, 0)

      for k in range(LOOK):
        prep_and_gather(base2, k, bufs[k])

      def phase(b, t, t_next):
        (srcb, dstb, gidx, eeb, invb, rows, sem_i, sem_d, sem_g) = t
        base = pl.multiple_of(base2 + b * EB, EB)

        @pl.when(b + LOOK < NB2)
        def _():
          prep_and_gather(base2, b + LOOK, t_next)

        pltpu.make_async_copy(xw_hbm.at[gidx], rows, sem_g).wait()
        pltpu.make_async_copy(
            ee_hbm.at[pl.ds(base, EB)], eeb, sem_d).wait()
        pltpu.make_async_copy(inv_hbm.at[dstb], invb, sem_d).wait()

        for g in range(ng):
          for h in range(heads):
            eev = plsc.load_gather(
                eeb, [g * LANE + iota, jnp.full((LANE,), h, _i32)])
            iv = plsc.load_gather(
                invb, [g * LANE + iota, jnp.full((LANE,), h, _i32)])
            wb[h, pl.ds(g * LANE, LANE)] = eev * iv

        def accgroup(hmap):
          heads_used = sorted(set(hmap))

          def run(g, _):
            goff = pl.multiple_of(g * LANE, LANE)
            dbase = (dstb[pl.ds(goff, LANE)] - lo) * ch
            wvs = {h: wb[h, pl.ds(goff, LANE)] for h in heads_used}
            for jj in range(LANE):
              j = goff + jj
              db = dbase[jj] + iota
              ws = {h: wvs[h][jj] for h in heads_used}
              for v in range(nv):
                plsc.addupdate_scatter(
                    acc, [db + v * LANE],
                    rows[j, pl.ds(v * LANE, LANE)] * ws[hmap[v]])
            return 0

          lax.fori_loop(0, ng, run, 0)

        @pl.when(c == 0)
        def _():
          accgroup(hmap0)

        @pl.when(c == 1)
        def _():
          accgroup(hmap1)

      def ring(i, _):
        for q in range(D):
          b = D * i + q
          phase(b, bufs[q], bufs[(q + LOOK) % D])
        return 0

      lax.fori_loop(0, NB2 // D, ring, 0)

      lof = pl.multiple_of(lo * ch, EB)

      @pl.when(c == 0)
      def _():
        pltpu.sync_copy(acc, out0_hbm.at[pl.ds(lof, BKT * ch)])

      @pl.when(c == 1)
      def _():
        pltpu.sync_copy(acc, out1_hbm.at[pl.ds(lof, BKT * ch)])

      return 0

    lax.fori_loop(0, 2, bucket, 0)

  dbuf = [
      pltpu.VMEM((EB,), _i32),
      pltpu.VMEM((EB,), _i32),
      pltpu.VMEM((EB,), _i32),
      pltpu.VMEM((EB, EW), _f32),
      pltpu.VMEM((EB, EW), _f32),
      pltpu.VMEM((EB, ch), _f32),
      pltpu.SemaphoreType.DMA,
      pltpu.SemaphoreType.DMA,
      pltpu.SemaphoreType.DMA,
  ]
  return pl.kernel(
      body,
      out_type=[jax.ShapeDtypeStruct((NPAD * ch,), _f32)] * 2,
      mesh=_mesh(),
      compiler_params=_SC_PARAMS,
      scratch_types=(
          dbuf * D
          + [pltpu.VMEM((8, EB), _f32),
             pltpu.VMEM((BKT * ch,), _f32)]),
  )


_HMAP_L1 = tuple(LANE * v // C1 for v in range(10))
_HMAP_L1_HI = tuple((160 + LANE * v) // C1 for v in range(10))
_HMAP_L2 = (0, 0, 0, 0)

_pass_a1 = _make_pass_a(H1)
_pass_a2 = _make_pass_a(1)
_pass_b1 = _make_pass_b(H1, 160, _HMAP_L1, _HMAP_L1_HI)
_pass_b2 = _make_pass_b(1, 64, _HMAP_L2, _HMAP_L2)


def _layer(sp, dp, cnts, xw, alpha_s, alpha_d, pass_a, pass_b, heads, ch):
  m = jnp.max(alpha_s, axis=0) + jnp.max(alpha_d, axis=0)
  cvec = jnp.maximum(m, NEG * m)
  cb = jnp.broadcast_to(cvec[:, None], (heads, LANE)).astype(_f32)
  ee, den = pass_a(sp, dp, cnts, alpha_s.ravel(), alpha_d.ravel(), cb)
  inv = _inv(den)
  xw_all = jnp.concatenate([xw[:, :ch], xw[:, ch:]], axis=0)
  o0, o1 = pass_b(sp, dp, ee, inv, xw_all)
  return o0.reshape(NPAD, ch)[:N], o1.reshape(NPAD, ch)[:N]


def kernel(x, edge_index, W1, a_src1, a_dst1, b1, W2, a_src2, a_dst2, b2):
  ei = edge_index.astype(_i32)
  loop = jnp.arange(N, dtype=_i32)
  padz = jnp.zeros((EP - ET,), _i32)
  src = jnp.concatenate([ei[0], loop, padz])
  dst = jnp.concatenate([ei[1], loop, padz])
  sp, dp, cnts = _partition(src, dst)

  xw1, as1, ad1 = _t1(x, W1, a_src1, a_dst1)
  d0, d1 = _layer(sp, dp, cnts, xw1, as1, ad1, _pass_a1, _pass_b1, H1, 160)
  xw2, as2, ad2 = _t2(d0, d1, b1, W2, a_src2, a_dst2)
  q0, q1 = _layer(sp, dp, cnts, xw2, as2, ad2, _pass_a2, _pass_b2, 1, 64)
  return _t3(q0, q1, b2)


# speedup vs baseline: 2.5479x; 2.5453x over previous
import functools

import jax
import jax.numpy as jnp
from jax import lax
from jax.experimental import pallas as pl
from jax.experimental.pallas import tpu as pltpu
from jax.experimental.pallas import tpu_sc as plsc

N = 10000
E = 320000
ET = E + N
D_IN = 128
H1 = 5
C1 = 64
HID = H1 * C1
D_OUT = 128
NEG = 0.2

NC = 2
NS = 16
LANE = 16

EB = 128
EP = 344064
NB_A = EP // (NC * NS * EB)
NB_B = EP // (NS * EB)
EW = 8
NPAD = 10240
ROWS_T = NPAD // NS

_f32 = jnp.float32
_i32 = jnp.int32


def _mesh():
  return plsc.VectorSubcoreMesh(
      core_axis_name="c", subcore_axis_name="s", num_cores=NC,
      num_subcores=NS)


def _iota16():
  return lax.iota(_i32, LANE)


def _t1_body(x_ref, w_ref, asr_ref, adr_ref, xw_ref, as_ref, ad_ref):
  xw = jnp.dot(x_ref[...], w_ref[...], preferred_element_type=_f32)
  xw_ref[...] = xw
  r = xw.shape[0]
  as_ref[...] = (xw * asr_ref[...]).reshape(r, H1, C1).sum(-1)
  ad_ref[...] = (xw * adr_ref[...]).reshape(r, H1, C1).sum(-1)


def _t1(x, W1, a_src1, a_dst1):
  blk = 1000
  grid = N // blk
  return pl.pallas_call(
      _t1_body,
      grid=(grid,),
      in_specs=[
          pl.BlockSpec((blk, D_IN), lambda i: (i, 0)),
          pl.BlockSpec((D_IN, HID), lambda i: (0, 0)),
          pl.BlockSpec((1, HID), lambda i: (0, 0)),
          pl.BlockSpec((1, HID), lambda i: (0, 0)),
      ],
      out_specs=[
          pl.BlockSpec((blk, HID), lambda i: (i, 0)),
          pl.BlockSpec((blk, H1), lambda i: (i, 0)),
          pl.BlockSpec((blk, H1), lambda i: (i, 0)),
      ],
      out_shape=[
          jax.ShapeDtypeStruct((N, HID), _f32),
          jax.ShapeDtypeStruct((N, H1), _f32),
          jax.ShapeDtypeStruct((N, H1), _f32),
      ],
  )(x, W1, a_src1.reshape(1, HID), a_dst1.reshape(1, HID))


def _t2_body(d0_ref, d1_ref, b1_ref, w_ref, asr_ref, adr_ref,
             xw_ref, as_ref, ad_ref):
  h0 = jnp.maximum(d0_ref[...] + b1_ref[0, :160], 0.0)
  h1 = jnp.maximum(d1_ref[...] + b1_ref[0, 160:], 0.0)
  xw = (jnp.dot(h0, w_ref[:160, :], preferred_element_type=_f32)
        + jnp.dot(h1, w_ref[160:, :], preferred_element_type=_f32))
  xw_ref[...] = xw
  as_ref[...] = (xw * asr_ref[...]).sum(-1, keepdims=True)
  ad_ref[...] = (xw * adr_ref[...]).sum(-1, keepdims=True)


def _t2(d0, d1, b1, W2, a_src2, a_dst2):
  blk = 1000
  grid = N // blk
  return pl.pallas_call(
      _t2_body,
      grid=(grid,),
      in_specs=[
          pl.BlockSpec((blk, 160), lambda i: (i, 0)),
          pl.BlockSpec((blk, 160), lambda i: (i, 0)),
          pl.BlockSpec((1, HID), lambda i: (0, 0)),
          pl.BlockSpec((HID, D_OUT), lambda i: (0, 0)),
          pl.BlockSpec((1, D_OUT), lambda i: (0, 0)),
          pl.BlockSpec((1, D_OUT), lambda i: (0, 0)),
      ],
      out_specs=[
          pl.BlockSpec((blk, D_OUT), lambda i: (i, 0)),
          pl.BlockSpec((blk, 1), lambda i: (i, 0)),
          pl.BlockSpec((blk, 1), lambda i: (i, 0)),
      ],
      out_shape=[
          jax.ShapeDtypeStruct((N, D_OUT), _f32),
          jax.ShapeDtypeStruct((N, 1), _f32),
          jax.ShapeDtypeStruct((N, 1), _f32),
      ],
  )(d0, d1, b1.reshape(1, HID), W2, a_src2.reshape(1, D_OUT),
    a_dst2.reshape(1, D_OUT))


def _inv_body(a_ref, b_ref, o_ref):
  o_ref[...] = 1.0 / (a_ref[...] + b_ref[...] + 1e-16)


def _inv(d2):
  a = d2[0, :N].reshape(625, 128)
  b = d2[1, :N].reshape(625, 128)
  out = pl.pallas_call(
      _inv_body,
      out_shape=jax.ShapeDtypeStruct((625, 128), _f32),
  )(a, b)
  return out.reshape(N, EW)


def _t3_body(q0_ref, q1_ref, b2_ref, o_ref):
  o_ref[...] = (jnp.concatenate([q0_ref[...], q1_ref[...]], axis=1)
                + b2_ref[...])


def _t3(q0, q1, b2):
  blk = 2000
  return pl.pallas_call(
      _t3_body,
      grid=(N // blk,),
      in_specs=[
          pl.BlockSpec((blk, 64), lambda i: (i, 0)),
          pl.BlockSpec((blk, 64), lambda i: (i, 0)),
          pl.BlockSpec((1, D_OUT), lambda i: (0, 0)),
      ],
      out_specs=pl.BlockSpec((blk, D_OUT), lambda i: (i, 0)),
      out_shape=jax.ShapeDtypeStruct((N, D_OUT), _f32),
  )(q0, q1, b2.reshape(1, D_OUT))


def _make_pass_a(heads):
  tab = N * heads

  def body(src_hbm, dst_hbm, as_hbm, ad_hbm, c_hbm,
           ee_hbm, den_hbm,
           as_v, ad_v, c_v, srcb, dstb, stage, den_sh):
    c = lax.axis_index("c")
    s = lax.axis_index("s")
    wid = c * NS + s
    pltpu.sync_copy(as_hbm, as_v)
    pltpu.sync_copy(ad_hbm, ad_v)
    pltpu.sync_copy(c_hbm, c_v)

    zeros = jnp.zeros((LANE,), _f32)

    iota0 = _iota16()
    zrow_r = iota0 // EW
    zrow_c = iota0 % EW

    def zrow(j, _):
      plsc.store_scatter(stage, [2 * j + zrow_r, zrow_c], zeros)
      return 0

    lax.fori_loop(0, EB // 2, zrow, 0)

    r0 = s * ROWS_T

    if True:
      for j in range(5):
        pltpu.sync_copy(stage, den_sh.at[pl.ds(r0 + j * EB, EB)])
      plsc.subcore_barrier()

      iota = _iota16()

      def batch(b, _):
        base = pl.multiple_of(wid * (NB_A * EB) + b * EB, EB)
        pltpu.sync_copy(src_hbm.at[pl.ds(base, EB)], srcb)
        pltpu.sync_copy(dst_hbm.at[pl.ds(base, EB)], dstb)
        for g in range(EB // LANE):
          sv = srcb[pl.ds(g * LANE, LANE)]
          dv = dstb[pl.ds(g * LANE, LANE)]
          k = base + g * LANE + iota
          live = k < ET
          for h in range(heads):
            av = plsc.load_gather(as_v, [sv * heads + h])
            bv = plsc.load_gather(ad_v, [dv * heads + h])
            e = av + bv
            e = jnp.maximum(e, NEG * e)
            ee = jnp.exp(e - c_v[h, :])
            ee = jnp.where(live, ee, 0.0)
            plsc.store_scatter(
                stage, [g * LANE + iota, jnp.full((LANE,), h, _i32)], ee)
        pltpu.sync_copy(stage, ee_hbm.at[pl.ds(base, EB)])
        pltpu.sync_copy(stage, den_sh.at[dstb], add=True)
        return 0

      lax.fori_loop(0, NB_A, batch, 0)
      plsc.subcore_barrier()

      @pl.when(c == 0)
      def _():
        pltpu.sync_copy(den_sh.at[pl.ds(r0, ROWS_T)],
                        den_hbm.at[0, pl.ds(r0, ROWS_T)])

      @pl.when(c == 1)
      def _():
        pltpu.sync_copy(den_sh.at[pl.ds(r0, ROWS_T)],
                        den_hbm.at[1, pl.ds(r0, ROWS_T)])


  k = pl.kernel(
      body,
      out_type=[
          jax.ShapeDtypeStruct((EP, EW), _f32),
          jax.ShapeDtypeStruct((2, NPAD, EW), _f32),
      ],
      mesh=_mesh(),
      compiler_params=pltpu.CompilerParams(needs_layout_passes=False, use_tc_tiling_on_sc=False),
      scratch_types=[
          pltpu.VMEM((tab,), _f32),
          pltpu.VMEM((tab,), _f32),
          pltpu.VMEM((heads, LANE), _f32),
          pltpu.VMEM((EB,), _i32),
          pltpu.VMEM((EB,), _i32),
          pltpu.VMEM((EB, EW), _f32),
          pltpu.VMEM_SHARED((NPAD, EW), _f32),
      ],
  )
  return k


def _make_pass_b(heads, ch, npass, hmaps, D=4, LOOK=2):
  nv = ch // LANE
  ng = EB // LANE

  def body(src_hbm, dst_hbm, ee_hbm, inv_hbm, *rest):
    xw_hbms = rest[:npass]
    out_hbms = rest[npass:npass + 2 * npass]
    rest = rest[3 * npass:]
    bufs = tuple(tuple(rest[11 * k: 11 * (k + 1)]) for k in range(D))
    wb, acc_sh = rest[11 * D:]
    c = lax.axis_index("c")
    s = lax.axis_index("s")

    zeros = jnp.zeros((LANE,), _f32)
    iota = _iota16()
    coff = c * N
    tb = s * (NB_B * EB)

    def base_of(b):
      return pl.multiple_of(tb + b * EB, EB)

    def prep_and_gather(b, t, xw_hbm):
      (srcb, dstb, gidx, eeb, invb, rows,
       sem_i, sem_d, sem_g, sem_v, sem_s) = t
      base = base_of(b)
      pltpu.async_copy(src_hbm.at[pl.ds(base, EB)], srcb, sem_i)
      pltpu.async_copy(dst_hbm.at[pl.ds(base, EB)], dstb, sem_i)
      pltpu.async_copy(ee_hbm.at[pl.ds(base, EB)], eeb, sem_d)
      pltpu.make_async_copy(
          src_hbm.at[pl.ds(base, EB)], srcb, sem_i).wait()
      pltpu.make_async_copy(
          dst_hbm.at[pl.ds(base, EB)], dstb, sem_i).wait()
      for g in range(ng):
        sv = srcb[pl.ds(g * LANE, LANE)]
        gidx[pl.ds(g * LANE, LANE)] = sv + coff
      pltpu.async_copy(xw_hbm.at[gidx], rows, sem_g)
      pltpu.async_copy(inv_hbm.at[dstb], invb, sem_v)

    def scale_loop(hmap, rows):
      heads_used = sorted(set(hmap))

      def sgroup(g, _):
        off = pl.multiple_of(g * LANE, LANE)
        wvs = {h: wb[h, pl.ds(off, LANE)] for h in heads_used}
        for jj in range(LANE):
          j = off + jj
          ws = {h: wvs[h][jj] for h in heads_used}
          for v in range(nv):
            rows[j, pl.ds(v * LANE, LANE)] = (
                rows[j, pl.ds(v * LANE, LANE)] * ws[hmap[v]])
        return 0

      lax.fori_loop(0, ng, sgroup, 0)

    def zrow(rows):
      def zr(j, _):
        for v in range(nv):
          rows[j, pl.ds(v * LANE, LANE)] = zeros
        return 0

      lax.fori_loop(0, EB, zr, 0)

    r0 = s * ROWS_T

    for p in range(npass):
      xw_hbm = xw_hbms[p]

      for t in bufs:
        zrow(t[5])
      for j in range(5):
        pltpu.sync_copy(bufs[0][5], acc_sh.at[pl.ds(r0 + j * EB, EB)])
      plsc.subcore_barrier()

      for k in range(LOOK):
        prep_and_gather(k, bufs[k], xw_hbm)

      def phase(b, t, t_next):
        (srcb, dstb, gidx, eeb, invb, rows,
         sem_i, sem_d, sem_g, sem_v, sem_s) = t
        n_dstb, n_rows, n_sem_s = t_next[1], t_next[5], t_next[10]
        base = base_of(b)

        @pl.when(b >= D - LOOK)
        def _():
          pltpu.make_async_copy(
              n_rows, acc_sh.at[n_dstb], n_sem_s).wait()

        @pl.when(b + LOOK < NB_B)
        def _():
          prep_and_gather(b + LOOK, t_next, xw_hbm)

        pltpu.make_async_copy(xw_hbm.at[gidx], rows, sem_g).wait()
        pltpu.make_async_copy(
            ee_hbm.at[pl.ds(base, EB)], eeb, sem_d).wait()
        pltpu.make_async_copy(inv_hbm.at[dstb], invb, sem_v).wait()

        for g in range(ng):
          for h in range(heads):
            eev = plsc.load_gather(
                eeb, [g * LANE + iota, jnp.full((LANE,), h, _i32)])
            iv = plsc.load_gather(
                invb, [g * LANE + iota, jnp.full((LANE,), h, _i32)])
            wb[h, pl.ds(g * LANE, LANE)] = eev * iv

        @pl.when(c == 0)
        def _():
          scale_loop(hmaps[(0, p)], rows)

        @pl.when(c == 1)
        def _():
          scale_loop(hmaps[(1, p)], rows)

        pltpu.async_copy(rows, acc_sh.at[dstb], sem_s, add=True)

      def ring(i, _):
        for q in range(D):
          b = D * i + q
          phase(b, bufs[q], bufs[(q + LOOK) % D])
        return 0

      lax.fori_loop(0, NB_B // D, ring, 0)

      for b in range(NB_B - LOOK, NB_B):
        t = bufs[b % D]
        pltpu.make_async_copy(t[5], acc_sh.at[t[1]], t[10]).wait()
      plsc.subcore_barrier()

      @pl.when(c == 0)
      def _():
        pltpu.sync_copy(acc_sh.at[pl.ds(r0, ROWS_T)],
                        out_hbms[p].at[pl.ds(r0, ROWS_T)])

      @pl.when(c == 1)
      def _():
        pltpu.sync_copy(acc_sh.at[pl.ds(r0, ROWS_T)],
                        out_hbms[npass + p].at[pl.ds(r0, ROWS_T)])

      if p + 1 < npass:
        plsc.subcore_barrier()

  dbuf = [
      pltpu.VMEM((EB,), _i32),
      pltpu.VMEM((EB,), _i32),
      pltpu.VMEM((EB,), _i32),
      pltpu.VMEM((EB, EW), _f32),
      pltpu.VMEM((EB, EW), _f32),
      pltpu.VMEM((EB, ch), _f32),
      pltpu.SemaphoreType.DMA,
      pltpu.SemaphoreType.DMA,
      pltpu.SemaphoreType.DMA,
      pltpu.SemaphoreType.DMA,
      pltpu.SemaphoreType.DMA,
  ]
  k = pl.kernel(
      body,
      out_type=[jax.ShapeDtypeStruct((NPAD, ch), _f32)] * (2 * npass),
      mesh=_mesh(),
      compiler_params=pltpu.CompilerParams(
          needs_layout_passes=False, use_tc_tiling_on_sc=False),
      scratch_types=(
          dbuf * D
          + [pltpu.VMEM((8, EB), _f32),
             pltpu.VMEM_SHARED((NPAD, ch), _f32)]),
  )
  return k


def _hmap(off, nv):
  return tuple((off + LANE * v) // C1 for v in range(nv))


_HMAPS_L1 = {(c, p): _hmap(c * 160 + p * 80, 5)
             for c in range(2) for p in range(2)}
_HMAPS_L2 = {(0, 0): (0, 0, 0, 0), (1, 0): (0, 0, 0, 0)}

_pass_a1 = _make_pass_a(H1)
_pass_a2 = _make_pass_a(1)
_pass_b1 = _make_pass_b(H1, 80, 2, _HMAPS_L1)
_pass_b2 = _make_pass_b(1, 64, 1, _HMAPS_L2)


def _layer(src, dst, xw, alpha_s, alpha_d, pass_a, pass_b, heads, chunks):
  m = jnp.max(alpha_s, axis=0) + jnp.max(alpha_d, axis=0)
  cvec = jnp.maximum(m, NEG * m)
  cb = jnp.broadcast_to(cvec[:, None], (heads, LANE)).astype(_f32)
  ee, den = pass_a(src, dst, alpha_s.ravel(), alpha_d.ravel(), cb)
  inv = _inv(den)
  npass = len(chunks) // 2
  xws = [jnp.concatenate([xw[:, chunks[p][0]:chunks[p][1]],
                          xw[:, chunks[npass + p][0]:chunks[npass + p][1]]],
                         axis=0)
         for p in range(npass)]
  outs = pass_b(src, dst, ee, inv, *xws)
  return [o[:N] for o in outs]


def kernel(x, edge_index, W1, a_src1, a_dst1, b1, W2, a_src2, a_dst2, b2):
  ei = edge_index.astype(_i32)
  loop = jnp.arange(N, dtype=_i32)
  padz = jnp.zeros((EP - ET,), _i32)
  src = jnp.concatenate([ei[0], loop, padz])
  dst = jnp.concatenate([ei[1], loop, padz])

  xw1, as1, ad1 = _t1(x, W1, a_src1, a_dst1)
  o = _layer(src, dst, xw1, as1, ad1, _pass_a1, _pass_b1, H1,
             [(0, 80), (80, 160), (160, 240), (240, 320)])
  d0 = jnp.concatenate([o[0], o[1]], axis=1)
  d1 = jnp.concatenate([o[2], o[3]], axis=1)
  xw2, as2, ad2 = _t2(d0, d1, b1, W2, a_src2, a_dst2)
  q = _layer(src, dst, xw2, as2, ad2, _pass_a2, _pass_b2, 1,
             [(0, 64), (64, 128)])
  return _t3(q[0], q[1], b2)
